# baseline (speedup 1.0000x reference)
import jax
import jax.numpy as jnp
from jax.experimental import pallas as pl
from jax.experimental.pallas import tpu as pltpu

NUM_CLS = 720
NUM_REG = 36
NUM_OUT = NUM_CLS + NUM_REG
FEAT_CH = 256


def _body(x_ref, w_ref, b_ref, cls_ref, reg_ref):
    blk_m = x_ref.shape[0]
    rows = reg_ref.shape[2]
    x = x_ref[...].astype(jnp.bfloat16)
    y = jax.lax.dot_general(
        x, w_ref[...].astype(jnp.bfloat16),
        dimension_numbers=(((1,), (1,)), ((), ())),
        preferred_element_type=jnp.float32,
    ) + b_ref[...]
    cls_ref[...] = y[:, :NUM_CLS]
    y2 = jnp.transpose(y[:, NUM_CLS:], (1, 0))
    reg_ref[0] = y2.reshape(NUM_REG, rows, blk_m // rows)


def kernel(feats, W_cls, b_cls, W_reg, b_reg):
    n, c, h, w = feats.shape
    m = n * h * w
    x = jnp.transpose(feats, (0, 2, 3, 1)).reshape(m, c)
    W = jnp.concatenate([W_cls, W_reg], axis=0)
    b = jnp.concatenate([b_cls, b_reg], axis=0).reshape(1, NUM_OUT)

    blk_m = 2048
    rows = blk_m // w
    nt = m // blk_m
    per_img = h // rows

    cls_y, reg_out = pl.pallas_call(
        _body,
        grid=(nt,),
        in_specs=[
            pl.BlockSpec((blk_m, c), lambda i: (i, 0)),
            pl.BlockSpec((NUM_OUT, c), lambda i: (0, 0)),
            pl.BlockSpec((1, NUM_OUT), lambda i: (0, 0)),
        ],
        out_specs=[
            pl.BlockSpec((blk_m, NUM_CLS), lambda i: (i, 0)),
            pl.BlockSpec((1, NUM_REG, rows, w),
                         lambda i: (i // per_img, 0, i % per_img, 0)),
        ],
        out_shape=[
            jax.ShapeDtypeStruct((m, NUM_CLS), jnp.float32),
            jax.ShapeDtypeStruct((n, NUM_REG, h, w), jnp.float32),
        ],
        compiler_params=pltpu.CompilerParams(
            dimension_semantics=("parallel",),
        ),
    )(x, W, b)

    cls_out = cls_y.reshape(n, h, w, NUM_CLS).transpose(0, 3, 1, 2)
    return (cls_out, reg_out)


# speedup vs baseline: 1.0758x; 1.0758x over previous
import jax
import jax.numpy as jnp
from jax.experimental import pallas as pl
from jax.experimental.pallas import tpu as pltpu

NUM_CLS = 720
NUM_REG = 36
FEAT_CH = 256


def _body(x_ref, wc_ref, bc_ref, wr_ref, br_ref, cls_ref, reg_ref):
    blk_m = x_ref.shape[0]
    rows = reg_ref.shape[2]
    x = x_ref[...].astype(jnp.bfloat16)
    y1 = jax.lax.dot_general(
        x, wc_ref[...].astype(jnp.bfloat16),
        dimension_numbers=(((1,), (1,)), ((), ())),
        preferred_element_type=jnp.float32,
    ) + bc_ref[...]
    cls_ref[...] = y1
    y2 = jax.lax.dot_general(
        wr_ref[...].astype(jnp.bfloat16), x,
        dimension_numbers=(((1,), (1,)), ((), ())),
        preferred_element_type=jnp.float32,
    ) + br_ref[...]
    reg_ref[0] = y2.reshape(NUM_REG, rows, blk_m // rows)


def kernel(feats, W_cls, b_cls, W_reg, b_reg):
    n, c, h, w = feats.shape
    m = n * h * w
    x = jnp.transpose(feats, (0, 2, 3, 1)).reshape(m, c)
    bc = b_cls.reshape(1, NUM_CLS)
    br = b_reg.reshape(NUM_REG, 1)

    blk_m = 4096
    rows = blk_m // w
    nt = m // blk_m
    per_img = h // rows

    cls_y, reg_out = pl.pallas_call(
        _body,
        grid=(nt,),
        in_specs=[
            pl.BlockSpec((blk_m, c), lambda i: (i, 0)),
            pl.BlockSpec((NUM_CLS, c), lambda i: (0, 0)),
            pl.BlockSpec((1, NUM_CLS), lambda i: (0, 0)),
            pl.BlockSpec((NUM_REG, c), lambda i: (0, 0)),
            pl.BlockSpec((NUM_REG, 1), lambda i: (0, 0)),
        ],
        out_specs=[
            pl.BlockSpec((blk_m, NUM_CLS), lambda i: (i, 0)),
            pl.BlockSpec((1, NUM_REG, rows, w),
                         lambda i: (i // per_img, 0, i % per_img, 0)),
        ],
        out_shape=[
            jax.ShapeDtypeStruct((m, NUM_CLS), jnp.float32),
            jax.ShapeDtypeStruct((n, NUM_REG, h, w), jnp.float32),
        ],
        compiler_params=pltpu.CompilerParams(
            dimension_semantics=("parallel",),
        ),
    )(x, W_cls, bc, W_reg, br)

    cls_out = cls_y.reshape(n, h, w, NUM_CLS).transpose(0, 3, 1, 2)
    return (cls_out, reg_out)


# speedup vs baseline: 1.1329x; 1.0531x over previous
import jax
import jax.numpy as jnp
from jax.experimental import pallas as pl
from jax.experimental.pallas import tpu as pltpu

NUM_CLS = 720
NUM_REG = 36
FEAT_CH = 256


def _body(x_ref, wc_ref, bc_ref, wr_ref, br_ref, cls_ref, reg_ref):
    blk_m = x_ref.shape[0]
    imgs, rows = reg_ref.shape[0], reg_ref.shape[2]
    x = x_ref[...].astype(jnp.bfloat16)
    y1 = jax.lax.dot_general(
        x, wc_ref[...].astype(jnp.bfloat16),
        dimension_numbers=(((1,), (1,)), ((), ())),
        preferred_element_type=jnp.float32,
    ) + bc_ref[...][None, :]
    cls_ref[...] = y1
    y2 = jax.lax.dot_general(
        wr_ref[...].astype(jnp.bfloat16), x,
        dimension_numbers=(((1,), (1,)), ((), ())),
        preferred_element_type=jnp.float32,
    ) + br_ref[...][:, None]
    reg_ref[...] = y2.reshape(NUM_REG, imgs, rows, blk_m // (imgs * rows)).transpose(1, 0, 2, 3)


def kernel(feats, W_cls, b_cls, W_reg, b_reg):
    n, c, h, w = feats.shape
    m = n * h * w
    x = jnp.transpose(feats, (0, 2, 3, 1)).reshape(m, c)

    blk_m = 4096
    imgs = blk_m // (h * w)
    rows = h
    nt = m // blk_m

    cls_y, reg_out = pl.pallas_call(
        _body,
        grid=(nt,),
        in_specs=[
            pl.BlockSpec((blk_m, c), lambda i: (i, 0)),
            pl.BlockSpec((NUM_CLS, c), lambda i: (0, 0)),
            pl.BlockSpec((NUM_CLS,), lambda i: (0,)),
            pl.BlockSpec((NUM_REG, c), lambda i: (0, 0)),
            pl.BlockSpec((NUM_REG,), lambda i: (0,)),
        ],
        out_specs=[
            pl.BlockSpec((blk_m, NUM_CLS), lambda i: (i, 0)),
            pl.BlockSpec((imgs, NUM_REG, rows, w),
                         lambda i: (i, 0, 0, 0)),
        ],
        out_shape=[
            jax.ShapeDtypeStruct((m, NUM_CLS), jnp.float32),
            jax.ShapeDtypeStruct((n, NUM_REG, h, w), jnp.float32),
        ],
        compiler_params=pltpu.CompilerParams(
            dimension_semantics=("parallel",),
        ),
    )(x, W_cls, b_cls, W_reg, b_reg)

    cls_out = cls_y.reshape(n, h, w, NUM_CLS).transpose(0, 3, 1, 2)
    return (cls_out, reg_out)
